# Initial kernel scaffold; baseline (speedup 1.0000x reference)
import jax
import jax.numpy as jnp
from jax.experimental import pallas as pl


def kernel(feat, edge_index, in_norm, out_norm, W, b):
    raise NotImplementedError("write your pallas kernel here")


# speedup vs baseline: 5.0644x; 5.0644x over previous
import functools

import jax
import jax.numpy as jnp
from jax import lax
from jax.experimental import pallas as pl
from jax.experimental.pallas import tpu as pltpu
from jax.experimental.pallas import tpu_sc as plsc

NC = 2
NS = 16
NW = NC * NS

CHUNK = 128
N_PAD = 10112
ROWS_PER_TILE = N_PAD // NS


def _prenorm_matmul_body(feat_ref, onorm_ref, w_ref, o_ref):
    h = feat_ref[...] * (1.0 / onorm_ref[...])
    o_ref[...] = jax.lax.dot_general(
        h, w_ref[...], (((1,), (1,)), ((), ())),
        preferred_element_type=jnp.float32)


def _postnorm_body(parts_ref, inorm_ref, b_ref, o_ref):
    s = parts_ref[0] + parts_ref[1]
    o_ref[...] = s * (1.0 / inorm_ref[...]) + b_ref[...]


def _edge_agg_body(src_hbm, dst_hbm, hp_hbm, zero_hbm, part_hbm,
                   src_v, dst_v, rows_v, agg_sh, sem):
    c = lax.axis_index("c")
    s = lax.axis_index("s")
    wid = c * NS + s

    t0 = s * ROWS_PER_TILE
    pltpu.sync_copy(zero_hbm.at[pl.ds(t0, ROWS_PER_TILE)],
                    agg_sh.at[pl.ds(t0, ROWS_PER_TILE)])

    pltpu.sync_copy(src_hbm.at[wid], src_v)
    pltpu.sync_copy(dst_hbm.at[wid], dst_v)
    plsc.subcore_barrier()

    n_chunks = src_hbm.shape[1]

    def body(j, carry):
        pltpu.async_copy(hp_hbm.at[src_v.at[j]], rows_v, sem).wait()
        pltpu.sync_copy(rows_v, agg_sh.at[dst_v.at[j]], add=True)
        return carry

    lax.fori_loop(0, n_chunks, body, 0)

    plsc.subcore_barrier()
    pltpu.sync_copy(agg_sh.at[pl.ds(t0, ROWS_PER_TILE)],
                    part_hbm.at[c, pl.ds(t0, ROWS_PER_TILE)])


def _edge_aggregate(src3, dst3, hp, zero):
    mesh = plsc.VectorSubcoreMesh(core_axis_name="c", subcore_axis_name="s")
    n_chunks = src3.shape[1]
    return pl.kernel(
        _edge_agg_body,
        out_type=jax.ShapeDtypeStruct((NC, N_PAD, 128), jnp.float32),
        mesh=mesh,
        scratch_types=[
            pltpu.VMEM((n_chunks, CHUNK), jnp.int32),
            pltpu.VMEM((n_chunks, CHUNK), jnp.int32),
            pltpu.VMEM((CHUNK, 128), jnp.float32),
            pltpu.VMEM_SHARED((N_PAD, 128), jnp.float32),
            pltpu.SemaphoreType.DMA,
        ],
    )(src3, dst3, hp, zero)


@jax.jit
def kernel(feat, edge_index, in_norm, out_norm, W, b):
    n, d_in = feat.shape
    e = edge_index.shape[1]

    pad_n = N_PAD - n
    feat_p = jnp.pad(feat, ((0, pad_n), (0, 0)))
    onorm_p = jnp.pad(out_norm, (0, pad_n), constant_values=1.0)[:, None]
    inorm_p = jnp.pad(in_norm, (0, pad_n), constant_values=1.0)[:, None]

    e_pad = -(-e // (NW * CHUNK)) * (NW * CHUNK)
    src = edge_index[0]
    dst = edge_index[1]
    npad_e = e_pad - e
    src_p = jnp.concatenate([src, jnp.zeros((npad_e,), jnp.int32)])
    dst_p = jnp.concatenate(
        [dst, n + (jnp.arange(npad_e, dtype=jnp.int32) % (N_PAD - n))])
    src3 = src_p.reshape(NW, -1, CHUNK)
    dst3 = dst_p.reshape(NW, -1, CHUNK)

    zero = jnp.zeros((N_PAD, 128), jnp.float32)

    hp = pl.pallas_call(
        _prenorm_matmul_body,
        out_shape=jax.ShapeDtypeStruct((N_PAD, 128), jnp.float32),
    )(feat_p, onorm_p, W)

    parts = _edge_aggregate(src3, dst3, hp, zero)

    out = pl.pallas_call(
        _postnorm_body,
        out_shape=jax.ShapeDtypeStruct((N_PAD, 128), jnp.float32),
    )(parts, inorm_p, b[None, :])

    return out[:n]
